# baseline (speedup 1.0000x reference)
import functools

import jax
import jax.numpy as jnp
from jax import lax
from jax.experimental import pallas as pl
from jax.experimental.pallas import tpu as pltpu
from jax.experimental.pallas import tpu_sc as plsc

_D = 32
_CHUNK = 128
_GROUPS = _CHUNK // 16


def _tec_body(cpw, nc, head_hbm, rel_hbm, tail_hbm, ent_hbm, rele_hbm, hyp_hbm,
              out_hbm, hidx, ridx, tidx, hrows, rrows, trows, wrows, outv, sem):
    wid = lax.axis_index("s") * nc + lax.axis_index("c")
    iota = lax.iota(jnp.int32, 16)

    def chunk_fn(j, carry):
        base = (wid * cpw + j) * _CHUNK
        pltpu.sync_copy(head_hbm.at[pl.ds(base, _CHUNK)], hidx)
        pltpu.sync_copy(rel_hbm.at[pl.ds(base, _CHUNK)], ridx)
        pltpu.sync_copy(tail_hbm.at[pl.ds(base, _CHUNK)], tidx)
        cp1 = pltpu.async_copy(ent_hbm.at[hidx], hrows, sem)
        cp2 = pltpu.async_copy(rele_hbm.at[ridx], rrows, sem)
        cp3 = pltpu.async_copy(ent_hbm.at[tidx], trows, sem)
        cp4 = pltpu.async_copy(hyp_hbm.at[ridx], wrows, sem)
        cp1.wait(); cp2.wait(); cp3.wait(); cp4.wait()

        def group_fn(g, gcarry):
            rowi = g * 16 + iota
            zero = jnp.zeros((16,), jnp.float32)
            s1 = zero; a = zero; bt = zero; cr = zero; s3 = zero
            for d in range(_D):
                col = jnp.full((16,), d, dtype=jnp.int32)
                h = plsc.load_gather(hrows, [rowi, col])
                r = plsc.load_gather(rrows, [rowi, col])
                t = plsc.load_gather(trows, [rowi, col])
                w = plsc.load_gather(wrows, [rowi, col])
                u = h + r - t
                s1 = s1 + u * u
                a = a + w * h
                bt = bt + w * t
                cr = cr + w * r
                s3 = s3 + w * w
            k = a - bt
            s2 = k + cr
            sq = s1 - 2.0 * k * s2 + k * k * s3
            sq = jnp.maximum(sq, 1e-30)
            ii = plsc.bitcast(sq, jnp.int32)
            yi = jnp.int32(0x5F3759DF) - (ii >> 1)
            y = plsc.bitcast(yi, jnp.float32)
            for _ in range(3):
                y = y * (1.5 - 0.5 * sq * y * y)
            outv[pl.ds(g * 16, 16)] = sq * y
            return gcarry

        lax.fori_loop(0, _GROUPS, group_fn, 0)
        pltpu.sync_copy(outv, out_hbm.at[pl.ds(base, _CHUNK)])
        return carry

    lax.fori_loop(0, cpw, chunk_fn, 0)


@functools.cache
def _build_sc_call(n_rows):
    mesh = plsc.VectorSubcoreMesh(core_axis_name="c", subcore_axis_name="s")
    nw = mesh.num_cores * mesh.num_subcores
    assert n_rows % (nw * _CHUNK) == 0
    cpw = n_rows // (nw * _CHUNK)
    return pl.kernel(
        functools.partial(_tec_body, cpw, mesh.num_cores),
        out_type=jax.ShapeDtypeStruct((n_rows,), jnp.float32),
        mesh=mesh,
        scratch_types=[
            pltpu.VMEM((_CHUNK,), jnp.int32),
            pltpu.VMEM((_CHUNK,), jnp.int32),
            pltpu.VMEM((_CHUNK,), jnp.int32),
            pltpu.VMEM((_CHUNK, _D), jnp.float32),
            pltpu.VMEM((_CHUNK, _D), jnp.float32),
            pltpu.VMEM((_CHUNK, _D), jnp.float32),
            pltpu.VMEM((_CHUNK, _D), jnp.float32),
            pltpu.VMEM((_CHUNK,), jnp.float32),
            pltpu.SemaphoreType.DMA,
        ],
        compiler_params=pltpu.CompilerParams(
            needs_layout_passes=False, use_tc_tiling_on_sc=False),
        name="transh_score_sc",
    )


def kernel(pos, neg, ent_embd, rel_embd, rel_hyper):
    b = pos.shape[1]
    head = jnp.concatenate([pos[0], neg[0]])
    rel = jnp.concatenate([pos[1], neg[1]])
    tail = jnp.concatenate([pos[2], neg[2]])
    scores = _build_sc_call(2 * b)(head, rel, tail, ent_embd, rel_embd,
                                   rel_hyper)
    pos_score = scores[:b].reshape(b, 1)
    neg_score = scores[b:].reshape(b, 1)
    return (pos_score, neg_score, ent_embd, rel_embd, rel_hyper)
